# baseline (speedup 1.0000x reference)
import functools
from math import sqrt

import jax
import jax.numpy as jnp
from jax import lax
from jax.experimental import pallas as pl
from jax.experimental.pallas import tpu as pltpu
from jax.experimental.pallas import tpu_sc as plsc

VOCAB_SIZE = 1000000
EMBEDDING_DIM = 64
SCALE = sqrt(EMBEDDING_DIM)

_INFO = plsc.get_sparse_core_info()
_NC = _INFO.num_cores
_NS = _INFO.num_subcores
_NW = _NC * _NS
_L = _INFO.num_lanes

_B = 4096 * 200
_PER_W = _B // _NW
_C = 512
_NCHUNK = _PER_W // _C


def _emb_body(table_hbm, idx_hbm, out_hbm, idx_v, rows_v, sem):
    wid = lax.axis_index("s") * _NC + lax.axis_index("c")
    base = wid * _PER_W

    def chunk(c, carry):
        off = base + c * _C
        pltpu.sync_copy(idx_hbm.at[pl.ds(off, _C)], idx_v)
        pltpu.async_copy(table_hbm.at[idx_v], rows_v, sem).wait()

        def scale_row(j, carry2):
            for k in range(EMBEDDING_DIM // _L):
                sl = pl.ds(k * _L, _L)
                rows_v[j, sl] = rows_v[j, sl] * SCALE
            return carry2

        lax.fori_loop(0, _C, scale_row, 0, unroll=4)
        pltpu.sync_copy(rows_v, out_hbm.at[pl.ds(off, _C)])
        return carry

    lax.fori_loop(0, _NCHUNK, chunk, 0)


@functools.partial(jax.jit, static_argnames=())
def _launch(idx, table):
    mesh = plsc.VectorSubcoreMesh(core_axis_name="c", subcore_axis_name="s")
    f = pl.kernel(
        _emb_body,
        mesh=mesh,
        out_type=jax.ShapeDtypeStruct((_B, EMBEDDING_DIM), jnp.float32),
        scratch_types=[
            pltpu.VMEM((_C,), jnp.int32),
            pltpu.VMEM((_C, EMBEDDING_DIM), jnp.float32),
            pltpu.SemaphoreType.DMA,
        ],
        compiler_params=pltpu.CompilerParams(use_tc_tiling_on_sc=False),
    )
    return f(table, idx)


def kernel(x, table):
    idx = x.reshape(-1).astype(jnp.int32)
    out = _launch(idx, table)
    return out.reshape(x.shape + (EMBEDDING_DIM,))


# speedup vs baseline: 1.0886x; 1.0886x over previous
import functools
from math import sqrt

import jax
import jax.numpy as jnp
from jax import lax
from jax.experimental import pallas as pl
from jax.experimental.pallas import tpu as pltpu
from jax.experimental.pallas import tpu_sc as plsc

VOCAB_SIZE = 1000000
EMBEDDING_DIM = 64
SCALE = sqrt(EMBEDDING_DIM)

_INFO = plsc.get_sparse_core_info()
_NC = _INFO.num_cores
_NS = _INFO.num_subcores
_NW = _NC * _NS
_L = _INFO.num_lanes

_B = 4096 * 200
_PER_W = _B // _NW
_C = 400
_NCHUNK = _PER_W // _C
_NBUF = 4
_LOOKAHEAD = 2


def _emb_body(table_hbm, idx_hbm, out_hbm, idx_all, rows,
              g0, g1, g2, g3, o0, o1, o2, o3):
    gsems = (g0, g1, g2, g3)
    osems = (o0, o1, o2, o3)
    wid = lax.axis_index("s") * _NC + lax.axis_index("c")
    base = wid * _PER_W

    pltpu.sync_copy(idx_hbm.at[pl.ds(base, _PER_W)], idx_all)

    def start_gather(g, b):
        pltpu.async_copy(
            table_hbm.at[idx_all.at[pl.ds(g * _C, _C)]], rows.at[b],
            gsems[b])

    def wait_gather(b):
        pltpu.make_async_copy(
            table_hbm.at[idx_all.at[pl.ds(0, _C)]], rows.at[b],
            gsems[b]).wait()

    def start_out(g, b):
        pltpu.async_copy(
            rows.at[b], out_hbm.at[pl.ds(base + g * _C, _C)], osems[b])

    def wait_out(b):
        pltpu.make_async_copy(
            rows.at[b], out_hbm.at[pl.ds(base, _C)], osems[b]).wait()

    for g in range(_LOOKAHEAD):
        start_gather(g, g % _NBUF)

    def step(i0, carry):
        for b in range(_NBUF):
            g = i0 * _NBUF + b
            bp = (b + _LOOKAHEAD) % _NBUF

            @pl.when(g + _LOOKAHEAD < _NCHUNK)
            def _():
                @pl.when(g >= _LOOKAHEAD)
                def _():
                    wait_out(bp)
                start_gather(g + _LOOKAHEAD, bp)

            wait_gather(b)

            @plsc.parallel_loop(0, _C, unroll=8)
            def _(j):
                for k in range(EMBEDDING_DIM // _L):
                    sl = pl.ds(k * _L, _L)
                    rows[b, j, sl] = rows[b, j, sl] * SCALE

            start_out(g, b)
        return carry

    lax.fori_loop(0, _NCHUNK // _NBUF, step, 0)

    for b in range(_NBUF):
        wait_out(b)


@jax.jit
def _launch(idx, table):
    mesh = plsc.VectorSubcoreMesh(core_axis_name="c", subcore_axis_name="s")
    f = pl.kernel(
        _emb_body,
        mesh=mesh,
        out_type=jax.ShapeDtypeStruct((_B, EMBEDDING_DIM), jnp.float32),
        scratch_types=[
            pltpu.VMEM((_PER_W,), jnp.int32),
            pltpu.VMEM((_NBUF, _C, EMBEDDING_DIM), jnp.float32),
        ] + [pltpu.SemaphoreType.DMA] * (2 * _NBUF) + [
        ],
        compiler_params=pltpu.CompilerParams(use_tc_tiling_on_sc=False),
    )
    return f(table, idx)


def kernel(x, table):
    idx = x.reshape(-1).astype(jnp.int32)
    out = _launch(idx, table)
    return out.reshape(x.shape + (EMBEDDING_DIM,))
